# baseline (speedup 1.0000x reference)
import functools

import jax
import jax.numpy as jnp
from jax import lax
from jax.experimental import pallas as pl
from jax.experimental.pallas import tpu as pltpu
from jax.experimental.pallas import tpu_sc as plsc

VOCAB = 1000000
DIM = 64
BATCH = 4096
SEQ = 200

NC = 2
NS = 16
NW = NC * NS
C = 128
B = BATCH * SEQ
NCH = B // (NW * C)
NBUF = 4
NGRP = NCH // NBUF


def _embed_body(idx_hbm, table_hbm, out_hbm, idx_v,
                rows0, rows1, rows2, rows3,
                g0, g1, g2, g3, w0, w1, w2, w3):
    rows = (rows0, rows1, rows2, rows3)
    gsem = (g0, g1, g2, g3)
    wsem = (w0, w1, w2, w3)
    wid = lax.axis_index("s") * NC + lax.axis_index("c")

    pltpu.sync_copy(idx_hbm.at[pl.ds(wid * NCH, NCH)], idx_v)

    def start_gather(c, b):
        pltpu.async_copy(table_hbm.at[idx_v.at[c]], rows[b], gsem[b])

    def wait_gather(b):
        pltpu.make_async_copy(table_hbm.at[idx_v.at[0]], rows[b], gsem[b]).wait()

    def start_write(c, b):
        base = (wid * NCH + c) * C
        pltpu.async_copy(rows[b], out_hbm.at[pl.ds(base, C)], wsem[b])

    def wait_write(b):
        base = wid * NCH * C
        pltpu.make_async_copy(rows[b], out_hbm.at[pl.ds(base, C)], wsem[b]).wait()

    for b in range(NBUF):
        start_gather(b, b)

    def run_group(g, restart):
        for b in range(NBUF):
            wait_gather(b)
            start_write(g * NBUF + b, b)
        if restart:
            for b in range(NBUF):
                wait_write(b)
                start_gather(g * NBUF + b + NBUF, b)

    def body(g, carry):
        run_group(g, True)
        return carry

    lax.fori_loop(0, NGRP - 1, body, 0)
    run_group(NGRP - 1, False)
    for b in range(NBUF):
        wait_write(b)


_embed_call = functools.partial(
    pl.kernel,
    mesh=plsc.VectorSubcoreMesh(core_axis_name="c", subcore_axis_name="s"),
    out_type=jax.ShapeDtypeStruct((B, DIM), jnp.float32),
    scratch_types=[
        pltpu.VMEM((NCH, C), jnp.int32),
        pltpu.VMEM((C, DIM), jnp.float32),
        pltpu.VMEM((C, DIM), jnp.float32),
        pltpu.VMEM((C, DIM), jnp.float32),
        pltpu.VMEM((C, DIM), jnp.float32),
        pltpu.SemaphoreType.DMA,
        pltpu.SemaphoreType.DMA,
        pltpu.SemaphoreType.DMA,
        pltpu.SemaphoreType.DMA,
        pltpu.SemaphoreType.DMA,
        pltpu.SemaphoreType.DMA,
        pltpu.SemaphoreType.DMA,
        pltpu.SemaphoreType.DMA,
    ],
    compiler_params=pltpu.CompilerParams(use_tc_tiling_on_sc=False),
)(_embed_body)


def kernel(token_ids, table):
    idx = token_ids.reshape(NW * NCH, C).astype(jnp.int32)
    out = _embed_call(idx, table)
    return out.reshape(BATCH, SEQ, DIM)


# speedup vs baseline: 1.0045x; 1.0045x over previous
import functools

import jax
import jax.numpy as jnp
from jax import lax
from jax.experimental import pallas as pl
from jax.experimental.pallas import tpu as pltpu
from jax.experimental.pallas import tpu_sc as plsc

VOCAB = 1000000
DIM = 64
BATCH = 4096
SEQ = 200

NC = 2
NS = 16
NW = NC * NS
C = 128
B = BATCH * SEQ
NCH = B // (NW * C)
D = 8
HALF = D // 2


def _embed_body(idx_hbm, table_hbm, out_hbm, idx_v, *bufs_and_sems):
    rows = bufs_and_sems[:D]
    gsem = bufs_and_sems[D:2 * D]
    wsem = bufs_and_sems[2 * D:3 * D]
    wid = lax.axis_index("s") * NC + lax.axis_index("c")

    pltpu.sync_copy(idx_hbm.at[pl.ds(wid * NCH, NCH)], idx_v)

    def start_gather(c, b):
        pltpu.async_copy(table_hbm.at[idx_v.at[c]], rows[b], gsem[b])

    def wait_gather(b):
        pltpu.make_async_copy(table_hbm.at[idx_v.at[0]], rows[b], gsem[b]).wait()

    def start_write(c, b):
        base = (wid * NCH + c) * C
        pltpu.async_copy(rows[b], out_hbm.at[pl.ds(base, C)], wsem[b])

    def wait_write(b):
        base = wid * NCH * C
        pltpu.make_async_copy(rows[b], out_hbm.at[pl.ds(base, C)], wsem[b]).wait()

    for t in range(HALF):
        start_gather(t, t)
    for t in range(HALF):
        wait_gather(t)
        start_write(t, t)
        start_gather(t + HALF, t + HALF)

    n_steady = (NCH - D) // D

    def body(k, carry):
        for j in range(D):
            t = HALF + k * D + j
            b = (HALF + j) % D
            wait_gather(b)
            start_write(t, b)
            b2 = (b + HALF) % D
            wait_write(b2)
            start_gather(t + HALF, b2)
        return carry

    lax.fori_loop(0, n_steady, body, 0)

    for j in range(HALF):
        t = NCH - HALF + j
        b = t % D
        wait_gather(b)
        start_write(t, b)
    for b in range(D):
        wait_write(b)


_embed_call = functools.partial(
    pl.kernel,
    mesh=plsc.VectorSubcoreMesh(core_axis_name="c", subcore_axis_name="s"),
    out_type=jax.ShapeDtypeStruct((B, DIM), jnp.float32),
    scratch_types=(
        [pltpu.VMEM((NCH, C), jnp.int32)]
        + [pltpu.VMEM((C, DIM), jnp.float32) for _ in range(D)]
        + [pltpu.SemaphoreType.DMA for _ in range(2 * D)]
    ),
    compiler_params=pltpu.CompilerParams(use_tc_tiling_on_sc=False),
)(_embed_body)


def kernel(token_ids, table):
    idx = token_ids.reshape(NW * NCH, C).astype(jnp.int32)
    out = _embed_call(idx, table)
    return out.reshape(BATCH, SEQ, DIM)


# speedup vs baseline: 1.0320x; 1.0274x over previous
import functools

import jax
import jax.numpy as jnp
from jax import lax
from jax.experimental import pallas as pl
from jax.experimental.pallas import tpu as pltpu
from jax.experimental.pallas import tpu_sc as plsc

VOCAB = 1000000
DIM = 64
BATCH = 4096
SEQ = 200

NC = 2
NS = 16
NW = NC * NS
C = BATCH // NW
NCH = SEQ
D = 8
HALF = D // 2


def _embed_body(idx_hbm, table_hbm, out_hbm, idx_v, *bufs_and_sems):
    rows = bufs_and_sems[:D]
    gsem = bufs_and_sems[D:2 * D]
    wsem = bufs_and_sems[2 * D:3 * D]
    wid = lax.axis_index("s") * NC + lax.axis_index("c")
    b0 = wid * C

    pltpu.sync_copy(idx_hbm.at[:, pl.ds(b0, C)], idx_v)

    def start_gather(s, b):
        pltpu.async_copy(table_hbm.at[idx_v.at[s]], rows[b], gsem[b])

    def wait_gather(b):
        pltpu.make_async_copy(table_hbm.at[idx_v.at[0]], rows[b], gsem[b]).wait()

    def start_write(s, b):
        pltpu.async_copy(rows[b], out_hbm.at[s, pl.ds(b0, C)], wsem[b])

    def wait_write(b):
        pltpu.make_async_copy(rows[b], out_hbm.at[0, pl.ds(b0, C)], wsem[b]).wait()

    for t in range(HALF):
        start_gather(t, t)
    for t in range(HALF):
        wait_gather(t)
        start_write(t, t)
        start_gather(t + HALF, t + HALF)

    n_steady = (NCH - D) // D

    def body(k, carry):
        for j in range(D):
            t = HALF + k * D + j
            b = (HALF + j) % D
            wait_gather(b)
            start_write(t, b)
            b2 = (b + HALF) % D
            wait_write(b2)
            start_gather(t + HALF, b2)
        return carry

    lax.fori_loop(0, n_steady, body, 0)

    for j in range(HALF):
        t = NCH - HALF + j
        b = t % D
        wait_gather(b)
        start_write(t, b)
    for b in range(D):
        wait_write(b)


_embed_call = functools.partial(
    pl.kernel,
    mesh=plsc.VectorSubcoreMesh(core_axis_name="c", subcore_axis_name="s"),
    out_type=jax.ShapeDtypeStruct((SEQ, BATCH, DIM), jnp.float32),
    scratch_types=(
        [pltpu.VMEM((NCH, C), jnp.int32)]
        + [pltpu.VMEM((C, DIM), jnp.float32) for _ in range(D)]
        + [pltpu.SemaphoreType.DMA for _ in range(2 * D)]
    ),
    compiler_params=pltpu.CompilerParams(use_tc_tiling_on_sc=False),
)(_embed_body)


def kernel(token_ids, table):
    idx_t = token_ids.T.astype(jnp.int32)
    out_t = _embed_call(idx_t, table)
    return out_t.transpose(1, 0, 2)
